# baseline (speedup 1.0000x reference)
import functools

import numpy as _np
import jax
import jax.numpy as jnp
from jax.experimental import pallas as pl
from jax.experimental.pallas import tpu as pltpu


def _block_compute(xv, w1_ref, b1_ref, w2_ref, b2_ref, w3_ref, b3_ref,
                   wd_ref, bd_ref, h1p_ref, stride):
    nb, H, W, Cin = xv.shape
    P = w1_ref.shape[1]
    Cout = w3_ref.shape[1]
    Ho, Wo = H // stride, W // stride
    M1 = nb * H * W
    M2 = nb * Ho * Wo

    h1 = jnp.dot(xv.reshape(M1, Cin), w1_ref[...],
                 preferred_element_type=jnp.float32)
    h1 = jnp.maximum(h1 + b1_ref[...], 0.0).astype(jnp.bfloat16)

    h1p_ref[...] = jnp.zeros_like(h1p_ref)
    h1p_ref[:, 1:H + 1, 1:W + 1, :] = h1.reshape(nb, H, W, P)

    acc = jnp.broadcast_to(b2_ref[...], (M2, P)).astype(jnp.float32)
    for t in range(9):
        di, dj = divmod(t, 3)
        v = h1p_ref[:, di:di + H, dj:dj + W, :]
        if stride == 2:
            v = v.reshape(nb, Ho, 2, Wo, 2, P)[:, :, 0, :, 0, :]
        acc = acc + jnp.dot(v.reshape(M2, P), w2_ref[t * P:(t + 1) * P, :],
                            preferred_element_type=jnp.float32)
    h2 = jnp.maximum(acc, 0.0).astype(jnp.bfloat16)

    y = jnp.dot(h2, w3_ref[...], preferred_element_type=jnp.float32) \
        + b3_ref[...]
    if wd_ref is not None:
        xd = xv
        if stride == 2:
            xd = xd.reshape(nb, Ho, 2, Wo, 2, Cin)[:, :, 0, :, 0, :]
        idn = jnp.dot(xd.reshape(M2, Cin), wd_ref[...],
                      preferred_element_type=jnp.float32) + bd_ref[...]
        idn = idn.astype(jnp.bfloat16)
        y = y + idn.astype(jnp.float32)
    else:
        y = y + xv.reshape(M2, Cout).astype(jnp.float32)
    return jnp.maximum(y, 0.0).astype(jnp.bfloat16).reshape(nb, Ho, Wo, Cout)


def _layer_body(*refs, nblocks, stride, H, W):
    x_ref = refs[0]
    idx = 1
    blk_refs = []
    for k in range(nblocks):
        n = 8 if k == 0 else 6
        blk_refs.append(refs[idx:idx + n])
        idx += n
    o_ref, h1p0_ref, h1p_ref = refs[idx], refs[idx + 1], refs[idx + 2]

    xv = x_ref[...]
    for k in range(nblocks):
        br = blk_refs[k]
        if k == 0:
            xv = _block_compute(xv, *br, h1p0_ref, stride)
        else:
            xv = _block_compute(xv, *br, None, None, h1p_ref, 1)
    o_ref[...] = xv


def _res_layer(x, blocks, *, stride, nb):
    N, H, W, Cin = x.shape
    P = blocks[0]["w1"].shape[1]
    Cout = blocks[0]["w3"].shape[1]
    Ho, Wo = H // stride, W // stride

    const2 = lambda i: (0, 0)
    args = [x]
    in_specs = [pl.BlockSpec((nb, H, W, Cin), lambda i: (i, 0, 0, 0))]
    for k, blk in enumerate(blocks):
        names = ["w1", "b1", "w2", "b2", "w3", "b3"]
        if k == 0:
            names += ["wd", "bd"]
        for nm in names:
            args.append(blk[nm])
            in_specs.append(pl.BlockSpec(blk[nm].shape, const2))

    return pl.pallas_call(
        functools.partial(_layer_body, nblocks=len(blocks), stride=stride,
                          H=H, W=W),
        out_shape=jax.ShapeDtypeStruct((N, Ho, Wo, Cout), jnp.bfloat16),
        grid=(N // nb,),
        in_specs=in_specs,
        out_specs=pl.BlockSpec((nb, Ho, Wo, Cout), lambda i: (i, 0, 0, 0)),
        scratch_shapes=[pltpu.VMEM((nb, H + 2, W + 2, P), jnp.bfloat16),
                        pltpu.VMEM((nb, Ho + 2, Wo + 2, P), jnp.bfloat16)],
        compiler_params=pltpu.CompilerParams(
            dimension_semantics=("parallel",),
            vmem_limit_bytes=100 * 1024 * 1024),
    )(*args)


def _patchify_body(x_ref, s_ref, o_ref):
    x = x_ref[...].reshape(3, 68, 2, 256)
    s = s_ref[...]
    for i in range(7):
        half = i % 2
        base = i // 2
        a = x[:, base:base + 64, half, :].reshape(192, 256)
        g = jnp.dot(a, s, preferred_element_type=jnp.float32)
        g = g.astype(jnp.bfloat16)
        for j in range(7):
            for ch in range(3):
                o_ref[0, i * 21 + j * 3 + ch] = \
                    g[ch * 64:(ch + 1) * 64, j * 128:j * 128 + 64]


def _patchify(xpad, sel):
    N = xpad.shape[0]
    const2 = lambda i: (0, 0)
    return pl.pallas_call(
        _patchify_body,
        out_shape=jax.ShapeDtypeStruct((N, 147, 64, 64), jnp.bfloat16),
        grid=(N,),
        in_specs=[
            pl.BlockSpec((1, 3, 136, 256), lambda i: (i, 0, 0, 0)),
            pl.BlockSpec(sel.shape, const2),
        ],
        out_specs=pl.BlockSpec((1, 147, 64, 64), lambda i: (i, 0, 0, 0)),
        compiler_params=pltpu.CompilerParams(
            dimension_semantics=("parallel",),
            vmem_limit_bytes=100 * 1024 * 1024),
    )(xpad, sel)


def _stem_body(p_ref, w_ref, b_ref, o_ref):
    K = p_ref.shape[1]
    C = w_ref.shape[1]
    a = p_ref[...].reshape(K, 4096)
    y = jax.lax.dot_general(a, w_ref[0:K, :], (((0,), (0,)), ((), ())),
                            preferred_element_type=jnp.float32)
    y = jnp.maximum(y + b_ref[...], 0.0).astype(jnp.bfloat16)
    y = y.reshape(64, 64, C)

    yr = y.reshape(32, 2, 64, C)
    even, odd = yr[:, 0], yr[:, 1]
    prev = jnp.concatenate([y[0:1], odd[:31]], axis=0)
    rp = jnp.maximum(jnp.maximum(even, odd), prev)

    rc = rp.reshape(32, 32, 2, C)
    evc, odc = rc[:, :, 0], rc[:, :, 1]
    prevc = jnp.concatenate([rp[:, 0:1], odc[:, :31]], axis=1)
    out = jnp.maximum(jnp.maximum(evc, odc), prevc)
    o_ref[...] = out.reshape(1, 32, 32, C)


def _stem(patches, w, b):
    N, K, _ = patches.shape
    C = w.shape[1]
    const2 = lambda i: (0, 0)
    return pl.pallas_call(
        _stem_body,
        out_shape=jax.ShapeDtypeStruct((N, 32, 32, C), jnp.bfloat16),
        grid=(N,),
        in_specs=[
            pl.BlockSpec((1, K, 4096), lambda i: (i, 0, 0)),
            pl.BlockSpec(w.shape, const2),
            pl.BlockSpec(b.shape, const2),
        ],
        out_specs=pl.BlockSpec((1, 32, 32, C), lambda i: (i, 0, 0, 0)),
        compiler_params=pltpu.CompilerParams(
            dimension_semantics=("parallel",),
            vmem_limit_bytes=100 * 1024 * 1024),
    )(patches, w, b)


def _fc_body(a_ref, w_ref, b_ref, o_ref, acc_ref):
    @pl.when(pl.program_id(1) == 0)
    def _():
        acc_ref[...] = jnp.zeros_like(acc_ref)

    acc_ref[...] += jnp.dot(a_ref[...], w_ref[...],
                            preferred_element_type=jnp.float32)

    @pl.when(pl.program_id(1) == pl.num_programs(1) - 1)
    def _():
        o_ref[...] = acc_ref[...] + b_ref[...]


def _fc(a, w, b, tk=4096, tn=128):
    M, K = a.shape
    _, Np = w.shape
    return pl.pallas_call(
        _fc_body,
        out_shape=jax.ShapeDtypeStruct((M, Np), jnp.float32),
        grid=(Np // tn, K // tk),
        in_specs=[
            pl.BlockSpec((M, tk), lambda j, k: (0, k)),
            pl.BlockSpec((tk, tn), lambda j, k: (k, j)),
            pl.BlockSpec((1, tn), lambda j, k: (0, j)),
        ],
        out_specs=pl.BlockSpec((M, tn), lambda j, k: (0, j)),
        scratch_shapes=[pltpu.VMEM((M, tn), jnp.float32)],
        compiler_params=pltpu.CompilerParams(
            dimension_semantics=("parallel", "arbitrary"),
            vmem_limit_bytes=100 * 1024 * 1024),
    )(a, w, b)


@jax.jit
def _forward(params, x_nchw):
    N = x_nchw.shape[0]

    xpad = jnp.pad(x_nchw.astype(jnp.bfloat16),
                   ((0, 0), (0, 0), (3, 5), (3, 125)))
    sel = _np.zeros((256, 896), _np.float32)
    for j in range(7):
        for c in range(64):
            sel[2 * c + j, j * 128 + c] = 1.0
    sel = jnp.asarray(sel, jnp.bfloat16)
    patches = _patchify(xpad, sel).reshape(N, 147, 4096)
    x = _stem(patches, params["conv1_w"], params["conv1_b"])

    nb_layer = {0: 2, 1: 4, 2: 8, 3: 16}
    for li, blocks in enumerate(params["layers"]):
        x = _res_layer(x, blocks, stride=1 if li == 0 else 2,
                       nb=nb_layer[li])

    feat = x.reshape(N, -1)
    y = _fc(feat, params["fc_w"], params["fc_b"])
    nrm = jnp.sqrt(jnp.sum(y * y, axis=-1, keepdims=True))
    return y / jnp.maximum(nrm, 1e-12) * 10.0


def kernel(conv1_w, conv1_b,
           L0b0_w1, L0b0_b1, L0b0_w2, L0b0_b2, L0b0_w3, L0b0_b3, L0b0_wd, L0b0_bd,
           L0b1_w1, L0b1_b1, L0b1_w2, L0b1_b2, L0b1_w3, L0b1_b3,
           L0b2_w1, L0b2_b1, L0b2_w2, L0b2_b2, L0b2_w3, L0b2_b3,
           L1b0_w1, L1b0_b1, L1b0_w2, L1b0_b2, L1b0_w3, L1b0_b3, L1b0_wd, L1b0_bd,
           L1b1_w1, L1b1_b1, L1b1_w2, L1b1_b2, L1b1_w3, L1b1_b3,
           L1b2_w1, L1b2_b1, L1b2_w2, L1b2_b2, L1b2_w3, L1b2_b3,
           L1b3_w1, L1b3_b1, L1b3_w2, L1b3_b2, L1b3_w3, L1b3_b3,
           L2b0_w1, L2b0_b1, L2b0_w2, L2b0_b2, L2b0_w3, L2b0_b3, L2b0_wd, L2b0_bd,
           L2b1_w1, L2b1_b1, L2b1_w2, L2b1_b2, L2b1_w3, L2b1_b3,
           L2b2_w1, L2b2_b1, L2b2_w2, L2b2_b2, L2b2_w3, L2b2_b3,
           L2b3_w1, L2b3_b1, L2b3_w2, L2b3_b2, L2b3_w3, L2b3_b3,
           L2b4_w1, L2b4_b1, L2b4_w2, L2b4_b2, L2b4_w3, L2b4_b3,
           L2b5_w1, L2b5_b1, L2b5_w2, L2b5_b2, L2b5_w3, L2b5_b3,
           L3b0_w1, L3b0_b1, L3b0_w2, L3b0_b2, L3b0_w3, L3b0_b3, L3b0_wd, L3b0_bd,
           L3b1_w1, L3b1_b1, L3b1_w2, L3b1_b2, L3b1_w3, L3b1_b3,
           L3b2_w1, L3b2_b1, L3b2_w2, L3b2_b2, L3b2_w3, L3b2_b3,
           fc_w, fc_b, x):
    params = {
        "conv1_w": conv1_w, "conv1_b": conv1_b,
        "fc_w": fc_w, "fc_b": fc_b,
        "layers": [
            [
                {"w1": L0b0_w1, "b1": L0b0_b1, "w2": L0b0_w2, "b2": L0b0_b2,
                 "w3": L0b0_w3, "b3": L0b0_b3, "wd": L0b0_wd, "bd": L0b0_bd},
                {"w1": L0b1_w1, "b1": L0b1_b1, "w2": L0b1_w2, "b2": L0b1_b2,
                 "w3": L0b1_w3, "b3": L0b1_b3},
                {"w1": L0b2_w1, "b1": L0b2_b1, "w2": L0b2_w2, "b2": L0b2_b2,
                 "w3": L0b2_w3, "b3": L0b2_b3},
            ],
            [
                {"w1": L1b0_w1, "b1": L1b0_b1, "w2": L1b0_w2, "b2": L1b0_b2,
                 "w3": L1b0_w3, "b3": L1b0_b3, "wd": L1b0_wd, "bd": L1b0_bd},
                {"w1": L1b1_w1, "b1": L1b1_b1, "w2": L1b1_w2, "b2": L1b1_b2,
                 "w3": L1b1_w3, "b3": L1b1_b3},
                {"w1": L1b2_w1, "b1": L1b2_b1, "w2": L1b2_w2, "b2": L1b2_b2,
                 "w3": L1b2_w3, "b3": L1b2_b3},
                {"w1": L1b3_w1, "b1": L1b3_b1, "w2": L1b3_w2, "b2": L1b3_b2,
                 "w3": L1b3_w3, "b3": L1b3_b3},
            ],
            [
                {"w1": L2b0_w1, "b1": L2b0_b1, "w2": L2b0_w2, "b2": L2b0_b2,
                 "w3": L2b0_w3, "b3": L2b0_b3, "wd": L2b0_wd, "bd": L2b0_bd},
                {"w1": L2b1_w1, "b1": L2b1_b1, "w2": L2b1_w2, "b2": L2b1_b2,
                 "w3": L2b1_w3, "b3": L2b1_b3},
                {"w1": L2b2_w1, "b1": L2b2_b1, "w2": L2b2_w2, "b2": L2b2_b2,
                 "w3": L2b2_w3, "b3": L2b2_b3},
                {"w1": L2b3_w1, "b1": L2b3_b1, "w2": L2b3_w2, "b2": L2b3_b2,
                 "w3": L2b3_w3, "b3": L2b3_b3},
                {"w1": L2b4_w1, "b1": L2b4_b1, "w2": L2b4_w2, "b2": L2b4_b2,
                 "w3": L2b4_w3, "b3": L2b4_b3},
                {"w1": L2b5_w1, "b1": L2b5_b1, "w2": L2b5_w2, "b2": L2b5_b2,
                 "w3": L2b5_w3, "b3": L2b5_b3},
            ],
            [
                {"w1": L3b0_w1, "b1": L3b0_b1, "w2": L3b0_w2, "b2": L3b0_b2,
                 "w3": L3b0_w3, "b3": L3b0_b3, "wd": L3b0_wd, "bd": L3b0_bd},
                {"w1": L3b1_w1, "b1": L3b1_b1, "w2": L3b1_w2, "b2": L3b1_b2,
                 "w3": L3b1_w3, "b3": L3b1_b3},
                {"w1": L3b2_w1, "b1": L3b2_b1, "w2": L3b2_w2, "b2": L3b2_b2,
                 "w3": L3b2_w3, "b3": L3b2_b3},
            ],
        ],
    }
    return _forward(params, x)


# speedup vs baseline: 1.0494x; 1.0494x over previous
import functools

import numpy as _np
import jax
import jax.numpy as jnp
from jax.experimental import pallas as pl
from jax.experimental.pallas import tpu as pltpu


def _block_compute(xv, w1_ref, b1_ref, w2_ref, b2_ref, w3_ref, b3_ref,
                   wd_ref, bd_ref, h1p_ref, stride):
    nb, H, W, Cin = xv.shape
    P = w1_ref.shape[1]
    Cout = w3_ref.shape[1]
    Ho, Wo = H // stride, W // stride
    M1 = nb * H * W
    M2 = nb * Ho * Wo

    h1 = jnp.dot(xv.reshape(M1, Cin), w1_ref[...],
                 preferred_element_type=jnp.float32)
    h1 = jnp.maximum(h1 + b1_ref[...], 0.0).astype(jnp.bfloat16)

    h1p_ref[...] = jnp.zeros_like(h1p_ref)
    h1p_ref[:, 1:H + 1, 1:W + 1, :] = h1.reshape(nb, H, W, P)

    taps = []
    for t in range(9):
        di, dj = divmod(t, 3)
        v = h1p_ref[:, di:di + H, dj:dj + W, :]
        if stride == 2:
            v = v.reshape(nb, Ho, 2, Wo, 2, P)[:, :, 0, :, 0, :]
        taps.append(v.reshape(M2, P))
    h2 = jnp.dot(jnp.concatenate(taps, axis=1), w2_ref[...],
                 preferred_element_type=jnp.float32) + b2_ref[...]
    h2 = jnp.maximum(h2, 0.0).astype(jnp.bfloat16)

    y = jnp.dot(h2, w3_ref[...], preferred_element_type=jnp.float32) \
        + b3_ref[...]
    if wd_ref is not None:
        xd = xv
        if stride == 2:
            xd = xd.reshape(nb, Ho, 2, Wo, 2, Cin)[:, :, 0, :, 0, :]
        idn = jnp.dot(xd.reshape(M2, Cin), wd_ref[...],
                      preferred_element_type=jnp.float32) + bd_ref[...]
        idn = idn.astype(jnp.bfloat16)
        y = y + idn.astype(jnp.float32)
    else:
        y = y + xv.reshape(M2, Cout).astype(jnp.float32)
    return jnp.maximum(y, 0.0).astype(jnp.bfloat16).reshape(nb, Ho, Wo, Cout)


def _layer_body(*refs, nblocks, stride, H, W):
    x_ref = refs[0]
    idx = 1
    blk_refs = []
    for k in range(nblocks):
        n = 8 if k == 0 else 6
        blk_refs.append(refs[idx:idx + n])
        idx += n
    o_ref, h1p0_ref, h1p_ref = refs[idx], refs[idx + 1], refs[idx + 2]

    xv = x_ref[...]
    for k in range(nblocks):
        br = blk_refs[k]
        if k == 0:
            xv = _block_compute(xv, *br, h1p0_ref, stride)
        else:
            xv = _block_compute(xv, *br, None, None, h1p_ref, 1)
    o_ref[...] = xv


def _res_layer(x, blocks, *, stride, nb):
    N, H, W, Cin = x.shape
    P = blocks[0]["w1"].shape[1]
    Cout = blocks[0]["w3"].shape[1]
    Ho, Wo = H // stride, W // stride

    const2 = lambda i: (0, 0)
    args = [x]
    in_specs = [pl.BlockSpec((nb, H, W, Cin), lambda i: (i, 0, 0, 0))]
    for k, blk in enumerate(blocks):
        names = ["w1", "b1", "w2", "b2", "w3", "b3"]
        if k == 0:
            names += ["wd", "bd"]
        for nm in names:
            args.append(blk[nm])
            in_specs.append(pl.BlockSpec(blk[nm].shape, const2))

    return pl.pallas_call(
        functools.partial(_layer_body, nblocks=len(blocks), stride=stride,
                          H=H, W=W),
        out_shape=jax.ShapeDtypeStruct((N, Ho, Wo, Cout), jnp.bfloat16),
        grid=(N // nb,),
        in_specs=in_specs,
        out_specs=pl.BlockSpec((nb, Ho, Wo, Cout), lambda i: (i, 0, 0, 0)),
        scratch_shapes=[pltpu.VMEM((nb, H + 2, W + 2, P), jnp.bfloat16),
                        pltpu.VMEM((nb, Ho + 2, Wo + 2, P), jnp.bfloat16)],
        compiler_params=pltpu.CompilerParams(
            dimension_semantics=("parallel",),
            vmem_limit_bytes=100 * 1024 * 1024),
    )(*args)


def _patchify_body(x_ref, s_ref, o_ref):
    x = x_ref[...].reshape(3, 68, 2, 256)
    s = s_ref[...]
    for i in range(7):
        half = i % 2
        base = i // 2
        a = x[:, base:base + 64, half, :].reshape(192, 256)
        g = jnp.dot(a, s, preferred_element_type=jnp.float32)
        g = g.astype(jnp.bfloat16)
        for j in range(7):
            for ch in range(3):
                o_ref[0, i * 21 + j * 3 + ch] = \
                    g[ch * 64:(ch + 1) * 64, j * 128:j * 128 + 64]


def _patchify(xpad, sel):
    N = xpad.shape[0]
    const2 = lambda i: (0, 0)
    return pl.pallas_call(
        _patchify_body,
        out_shape=jax.ShapeDtypeStruct((N, 147, 64, 64), jnp.bfloat16),
        grid=(N,),
        in_specs=[
            pl.BlockSpec((1, 3, 136, 256), lambda i: (i, 0, 0, 0)),
            pl.BlockSpec(sel.shape, const2),
        ],
        out_specs=pl.BlockSpec((1, 147, 64, 64), lambda i: (i, 0, 0, 0)),
        compiler_params=pltpu.CompilerParams(
            dimension_semantics=("parallel",),
            vmem_limit_bytes=100 * 1024 * 1024),
    )(xpad, sel)


def _stem_body(p_ref, w_ref, b_ref, o_ref):
    K = p_ref.shape[1]
    C = w_ref.shape[1]
    a = p_ref[...].reshape(K, 4096)
    y = jax.lax.dot_general(a, w_ref[0:K, :], (((0,), (0,)), ((), ())),
                            preferred_element_type=jnp.float32)
    y = jnp.maximum(y + b_ref[...], 0.0).astype(jnp.bfloat16)
    y = y.reshape(64, 64, C)

    yr = y.reshape(32, 2, 64, C)
    even, odd = yr[:, 0], yr[:, 1]
    prev = jnp.concatenate([y[0:1], odd[:31]], axis=0)
    rp = jnp.maximum(jnp.maximum(even, odd), prev)

    rc = rp.reshape(32, 32, 2, C)
    evc, odc = rc[:, :, 0], rc[:, :, 1]
    prevc = jnp.concatenate([rp[:, 0:1], odc[:, :31]], axis=1)
    out = jnp.maximum(jnp.maximum(evc, odc), prevc)
    o_ref[...] = out.reshape(1, 32, 32, C)


def _stem(patches, w, b):
    N, K, _ = patches.shape
    C = w.shape[1]
    const2 = lambda i: (0, 0)
    return pl.pallas_call(
        _stem_body,
        out_shape=jax.ShapeDtypeStruct((N, 32, 32, C), jnp.bfloat16),
        grid=(N,),
        in_specs=[
            pl.BlockSpec((1, K, 4096), lambda i: (i, 0, 0)),
            pl.BlockSpec(w.shape, const2),
            pl.BlockSpec(b.shape, const2),
        ],
        out_specs=pl.BlockSpec((1, 32, 32, C), lambda i: (i, 0, 0, 0)),
        compiler_params=pltpu.CompilerParams(
            dimension_semantics=("parallel",),
            vmem_limit_bytes=100 * 1024 * 1024),
    )(patches, w, b)


def _fc_body(a_ref, w_ref, b_ref, o_ref, acc_ref):
    @pl.when(pl.program_id(1) == 0)
    def _():
        acc_ref[...] = jnp.zeros_like(acc_ref)

    acc_ref[...] += jnp.dot(a_ref[...], w_ref[...],
                            preferred_element_type=jnp.float32)

    @pl.when(pl.program_id(1) == pl.num_programs(1) - 1)
    def _():
        o_ref[...] = acc_ref[...] + b_ref[...]


def _fc(a, w, b, tk=4096, tn=128):
    M, K = a.shape
    _, Np = w.shape
    return pl.pallas_call(
        _fc_body,
        out_shape=jax.ShapeDtypeStruct((M, Np), jnp.float32),
        grid=(Np // tn, K // tk),
        in_specs=[
            pl.BlockSpec((M, tk), lambda j, k: (0, k)),
            pl.BlockSpec((tk, tn), lambda j, k: (k, j)),
            pl.BlockSpec((1, tn), lambda j, k: (0, j)),
        ],
        out_specs=pl.BlockSpec((M, tn), lambda j, k: (0, j)),
        scratch_shapes=[pltpu.VMEM((M, tn), jnp.float32)],
        compiler_params=pltpu.CompilerParams(
            dimension_semantics=("parallel", "arbitrary"),
            vmem_limit_bytes=100 * 1024 * 1024),
    )(a, w, b)


@jax.jit
def _forward(params, x_nchw):
    N = x_nchw.shape[0]

    xpad = jnp.pad(x_nchw.astype(jnp.bfloat16),
                   ((0, 0), (0, 0), (3, 5), (3, 125)))
    sel = _np.zeros((256, 896), _np.float32)
    for j in range(7):
        for c in range(64):
            sel[2 * c + j, j * 128 + c] = 1.0
    sel = jnp.asarray(sel, jnp.bfloat16)
    patches = _patchify(xpad, sel).reshape(N, 147, 4096)
    x = _stem(patches, params["conv1_w"], params["conv1_b"])

    nb_layer = {0: 2, 1: 4, 2: 8, 3: 16}
    for li, blocks in enumerate(params["layers"]):
        x = _res_layer(x, blocks, stride=1 if li == 0 else 2,
                       nb=nb_layer[li])

    feat = x.reshape(N, -1)
    y = _fc(feat, params["fc_w"], params["fc_b"])
    nrm = jnp.sqrt(jnp.sum(y * y, axis=-1, keepdims=True))
    return y / jnp.maximum(nrm, 1e-12) * 10.0


def kernel(conv1_w, conv1_b,
           L0b0_w1, L0b0_b1, L0b0_w2, L0b0_b2, L0b0_w3, L0b0_b3, L0b0_wd, L0b0_bd,
           L0b1_w1, L0b1_b1, L0b1_w2, L0b1_b2, L0b1_w3, L0b1_b3,
           L0b2_w1, L0b2_b1, L0b2_w2, L0b2_b2, L0b2_w3, L0b2_b3,
           L1b0_w1, L1b0_b1, L1b0_w2, L1b0_b2, L1b0_w3, L1b0_b3, L1b0_wd, L1b0_bd,
           L1b1_w1, L1b1_b1, L1b1_w2, L1b1_b2, L1b1_w3, L1b1_b3,
           L1b2_w1, L1b2_b1, L1b2_w2, L1b2_b2, L1b2_w3, L1b2_b3,
           L1b3_w1, L1b3_b1, L1b3_w2, L1b3_b2, L1b3_w3, L1b3_b3,
           L2b0_w1, L2b0_b1, L2b0_w2, L2b0_b2, L2b0_w3, L2b0_b3, L2b0_wd, L2b0_bd,
           L2b1_w1, L2b1_b1, L2b1_w2, L2b1_b2, L2b1_w3, L2b1_b3,
           L2b2_w1, L2b2_b1, L2b2_w2, L2b2_b2, L2b2_w3, L2b2_b3,
           L2b3_w1, L2b3_b1, L2b3_w2, L2b3_b2, L2b3_w3, L2b3_b3,
           L2b4_w1, L2b4_b1, L2b4_w2, L2b4_b2, L2b4_w3, L2b4_b3,
           L2b5_w1, L2b5_b1, L2b5_w2, L2b5_b2, L2b5_w3, L2b5_b3,
           L3b0_w1, L3b0_b1, L3b0_w2, L3b0_b2, L3b0_w3, L3b0_b3, L3b0_wd, L3b0_bd,
           L3b1_w1, L3b1_b1, L3b1_w2, L3b1_b2, L3b1_w3, L3b1_b3,
           L3b2_w1, L3b2_b1, L3b2_w2, L3b2_b2, L3b2_w3, L3b2_b3,
           fc_w, fc_b, x):
    params = {
        "conv1_w": conv1_w, "conv1_b": conv1_b,
        "fc_w": fc_w, "fc_b": fc_b,
        "layers": [
            [
                {"w1": L0b0_w1, "b1": L0b0_b1, "w2": L0b0_w2, "b2": L0b0_b2,
                 "w3": L0b0_w3, "b3": L0b0_b3, "wd": L0b0_wd, "bd": L0b0_bd},
                {"w1": L0b1_w1, "b1": L0b1_b1, "w2": L0b1_w2, "b2": L0b1_b2,
                 "w3": L0b1_w3, "b3": L0b1_b3},
                {"w1": L0b2_w1, "b1": L0b2_b1, "w2": L0b2_w2, "b2": L0b2_b2,
                 "w3": L0b2_w3, "b3": L0b2_b3},
            ],
            [
                {"w1": L1b0_w1, "b1": L1b0_b1, "w2": L1b0_w2, "b2": L1b0_b2,
                 "w3": L1b0_w3, "b3": L1b0_b3, "wd": L1b0_wd, "bd": L1b0_bd},
                {"w1": L1b1_w1, "b1": L1b1_b1, "w2": L1b1_w2, "b2": L1b1_b2,
                 "w3": L1b1_w3, "b3": L1b1_b3},
                {"w1": L1b2_w1, "b1": L1b2_b1, "w2": L1b2_w2, "b2": L1b2_b2,
                 "w3": L1b2_w3, "b3": L1b2_b3},
                {"w1": L1b3_w1, "b1": L1b3_b1, "w2": L1b3_w2, "b2": L1b3_b2,
                 "w3": L1b3_w3, "b3": L1b3_b3},
            ],
            [
                {"w1": L2b0_w1, "b1": L2b0_b1, "w2": L2b0_w2, "b2": L2b0_b2,
                 "w3": L2b0_w3, "b3": L2b0_b3, "wd": L2b0_wd, "bd": L2b0_bd},
                {"w1": L2b1_w1, "b1": L2b1_b1, "w2": L2b1_w2, "b2": L2b1_b2,
                 "w3": L2b1_w3, "b3": L2b1_b3},
                {"w1": L2b2_w1, "b1": L2b2_b1, "w2": L2b2_w2, "b2": L2b2_b2,
                 "w3": L2b2_w3, "b3": L2b2_b3},
                {"w1": L2b3_w1, "b1": L2b3_b1, "w2": L2b3_w2, "b2": L2b3_b2,
                 "w3": L2b3_w3, "b3": L2b3_b3},
                {"w1": L2b4_w1, "b1": L2b4_b1, "w2": L2b4_w2, "b2": L2b4_b2,
                 "w3": L2b4_w3, "b3": L2b4_b3},
                {"w1": L2b5_w1, "b1": L2b5_b1, "w2": L2b5_w2, "b2": L2b5_b2,
                 "w3": L2b5_w3, "b3": L2b5_b3},
            ],
            [
                {"w1": L3b0_w1, "b1": L3b0_b1, "w2": L3b0_w2, "b2": L3b0_b2,
                 "w3": L3b0_w3, "b3": L3b0_b3, "wd": L3b0_wd, "bd": L3b0_bd},
                {"w1": L3b1_w1, "b1": L3b1_b1, "w2": L3b1_w2, "b2": L3b1_b2,
                 "w3": L3b1_w3, "b3": L3b1_b3},
                {"w1": L3b2_w1, "b1": L3b2_b1, "w2": L3b2_w2, "b2": L3b2_b2,
                 "w3": L3b2_w3, "b3": L3b2_b3},
            ],
        ],
    }
    return _forward(params, x)


# speedup vs baseline: 1.0500x; 1.0006x over previous
import functools

import numpy as _np
import jax
import jax.numpy as jnp
from jax.experimental import pallas as pl
from jax.experimental.pallas import tpu as pltpu


def _round8(x):
    return (x + 7) // 8 * 8


def _block_compute(xv, w1_ref, b1_ref, w2_ref, b2_ref, w3_ref, b3_ref,
                   wd_ref, bd_ref, h1p_ref, stride):
    nb, H, W, Cin = xv.shape
    P = w1_ref.shape[1]
    Cout = w3_ref.shape[1]
    Ho, Wo = H // stride, W // stride
    M1 = nb * H * W
    M2 = nb * Ho * Wo

    h1 = jnp.dot(xv.reshape(M1, Cin), w1_ref[...],
                 preferred_element_type=jnp.float32)
    h1 = jnp.maximum(h1 + b1_ref[...], 0.0).astype(jnp.bfloat16)

    h1p_ref[...] = jnp.zeros_like(h1p_ref)
    h1p_ref[:, 1:H + 1, 1:W + 1, :] = h1.reshape(nb, H, W, P)

    taps = []
    for t in range(9):
        di, dj = divmod(t, 3)
        v = h1p_ref[:, di:di + H, dj:dj + W, :]
        if stride == 2:
            v = v.reshape(nb, Ho, 2, Wo, 2, P)[:, :, 0, :, 0, :]
        taps.append(v.reshape(M2, P))
    h2 = jnp.dot(jnp.concatenate(taps, axis=1), w2_ref[...],
                 preferred_element_type=jnp.float32) + b2_ref[...]
    h2 = jnp.maximum(h2, 0.0).astype(jnp.bfloat16)

    y = jnp.dot(h2, w3_ref[...], preferred_element_type=jnp.float32) \
        + b3_ref[...]
    if wd_ref is not None:
        xd = xv
        if stride == 2:
            xd = xd.reshape(nb, Ho, 2, Wo, 2, Cin)[:, :, 0, :, 0, :]
        idn = jnp.dot(xd.reshape(M2, Cin), wd_ref[...],
                      preferred_element_type=jnp.float32) + bd_ref[...]
        idn = idn.astype(jnp.bfloat16)
        y = y + idn.astype(jnp.float32)
    else:
        y = y + xv.reshape(M2, Cout).astype(jnp.float32)
    return jnp.maximum(y, 0.0).astype(jnp.bfloat16).reshape(nb, Ho, Wo, Cout)


def _layer_body(*refs, nblocks, stride, H, W):
    x_ref = refs[0]
    idx = 1
    blk_refs = []
    for k in range(nblocks):
        n = 8 if k == 0 else 6
        blk_refs.append(refs[idx:idx + n])
        idx += n
    o_ref, h1p0_ref, h1p_ref = refs[idx], refs[idx + 1], refs[idx + 2]

    xv = x_ref[...]
    for k in range(nblocks):
        br = blk_refs[k]
        if k == 0:
            xv = _block_compute(xv, *br, h1p0_ref, stride)
        else:
            xv = _block_compute(xv, *br, None, None, h1p_ref, 1)
    o_ref[...] = xv


def _res_layer(x, blocks, *, stride, nb):
    N, H, W, Cin = x.shape
    P = blocks[0]["w1"].shape[1]
    Cout = blocks[0]["w3"].shape[1]
    Ho, Wo = H // stride, W // stride

    const2 = lambda i: (0, 0)
    args = [x]
    in_specs = [pl.BlockSpec((nb, H, W, Cin), lambda i: (i, 0, 0, 0))]
    for k, blk in enumerate(blocks):
        names = ["w1", "b1", "w2", "b2", "w3", "b3"]
        if k == 0:
            names += ["wd", "bd"]
        for nm in names:
            args.append(blk[nm])
            in_specs.append(pl.BlockSpec(blk[nm].shape, const2))

    return pl.pallas_call(
        functools.partial(_layer_body, nblocks=len(blocks), stride=stride,
                          H=H, W=W),
        out_shape=jax.ShapeDtypeStruct((N, Ho, Wo, Cout), jnp.bfloat16),
        grid=(N // nb,),
        in_specs=in_specs,
        out_specs=pl.BlockSpec((nb, Ho, Wo, Cout), lambda i: (i, 0, 0, 0)),
        scratch_shapes=[pltpu.VMEM((nb, H + 2, _round8(W + 2), P),
                                   jnp.bfloat16),
                        pltpu.VMEM((nb, Ho + 2, _round8(Wo + 2), P),
                                   jnp.bfloat16)],
        compiler_params=pltpu.CompilerParams(
            dimension_semantics=("parallel",),
            vmem_limit_bytes=100 * 1024 * 1024),
    )(*args)


def _patchify_body(x_ref, s_ref, o_ref):
    x = x_ref[...].reshape(3, 68, 2, 256)
    s = s_ref[...]
    for i in range(7):
        half = i % 2
        base = i // 2
        a = x[:, base:base + 64, half, :].reshape(192, 256)
        g = jnp.dot(a, s, preferred_element_type=jnp.float32)
        g = g.astype(jnp.bfloat16)
        for j in range(7):
            for ch in range(3):
                o_ref[0, i * 21 + j * 3 + ch] = \
                    g[ch * 64:(ch + 1) * 64, j * 128:j * 128 + 64]


def _patchify(xpad, sel):
    N = xpad.shape[0]
    const2 = lambda i: (0, 0)
    return pl.pallas_call(
        _patchify_body,
        out_shape=jax.ShapeDtypeStruct((N, 147, 64, 64), jnp.bfloat16),
        grid=(N,),
        in_specs=[
            pl.BlockSpec((1, 3, 136, 256), lambda i: (i, 0, 0, 0)),
            pl.BlockSpec(sel.shape, const2),
        ],
        out_specs=pl.BlockSpec((1, 147, 64, 64), lambda i: (i, 0, 0, 0)),
        compiler_params=pltpu.CompilerParams(
            dimension_semantics=("parallel",),
            vmem_limit_bytes=100 * 1024 * 1024),
    )(xpad, sel)


def _stem_body(p_ref, w_ref, b_ref, o_ref):
    K = p_ref.shape[1]
    C = w_ref.shape[1]
    a = p_ref[...].reshape(K, 4096)
    y = jax.lax.dot_general(a, w_ref[0:K, :], (((0,), (0,)), ((), ())),
                            preferred_element_type=jnp.float32)
    y = jnp.maximum(y + b_ref[...], 0.0).astype(jnp.bfloat16)
    y = y.reshape(64, 64, C)

    yr = y.reshape(32, 2, 64, C)
    even, odd = yr[:, 0], yr[:, 1]
    prev = jnp.concatenate([y[0:1], odd[:31]], axis=0)
    rp = jnp.maximum(jnp.maximum(even, odd), prev)

    rc = rp.reshape(32, 32, 2, C)
    evc, odc = rc[:, :, 0], rc[:, :, 1]
    prevc = jnp.concatenate([rp[:, 0:1], odc[:, :31]], axis=1)
    out = jnp.maximum(jnp.maximum(evc, odc), prevc)
    o_ref[...] = out.reshape(1, 32, 32, C)


def _stem(patches, w, b):
    N, K, _ = patches.shape
    C = w.shape[1]
    const2 = lambda i: (0, 0)
    return pl.pallas_call(
        _stem_body,
        out_shape=jax.ShapeDtypeStruct((N, 32, 32, C), jnp.bfloat16),
        grid=(N,),
        in_specs=[
            pl.BlockSpec((1, K, 4096), lambda i: (i, 0, 0)),
            pl.BlockSpec(w.shape, const2),
            pl.BlockSpec(b.shape, const2),
        ],
        out_specs=pl.BlockSpec((1, 32, 32, C), lambda i: (i, 0, 0, 0)),
        compiler_params=pltpu.CompilerParams(
            dimension_semantics=("parallel",),
            vmem_limit_bytes=100 * 1024 * 1024),
    )(patches, w, b)


def _fc_body(a_ref, w_ref, b_ref, o_ref, acc_ref):
    @pl.when(pl.program_id(1) == 0)
    def _():
        acc_ref[...] = jnp.zeros_like(acc_ref)

    acc_ref[...] += jnp.dot(a_ref[...], w_ref[...],
                            preferred_element_type=jnp.float32)

    @pl.when(pl.program_id(1) == pl.num_programs(1) - 1)
    def _():
        o_ref[...] = acc_ref[...] + b_ref[...]


def _fc(a, w, b, tk=4096, tn=128):
    M, K = a.shape
    _, Np = w.shape
    return pl.pallas_call(
        _fc_body,
        out_shape=jax.ShapeDtypeStruct((M, Np), jnp.float32),
        grid=(Np // tn, K // tk),
        in_specs=[
            pl.BlockSpec((M, tk), lambda j, k: (0, k)),
            pl.BlockSpec((tk, tn), lambda j, k: (k, j)),
            pl.BlockSpec((1, tn), lambda j, k: (0, j)),
        ],
        out_specs=pl.BlockSpec((M, tn), lambda j, k: (0, j)),
        scratch_shapes=[pltpu.VMEM((M, tn), jnp.float32)],
        compiler_params=pltpu.CompilerParams(
            dimension_semantics=("parallel", "arbitrary"),
            vmem_limit_bytes=100 * 1024 * 1024),
    )(a, w, b)


@jax.jit
def _forward(params, x_nchw):
    N = x_nchw.shape[0]

    xpad = jnp.pad(x_nchw.astype(jnp.bfloat16),
                   ((0, 0), (0, 0), (3, 5), (3, 125)))
    sel = _np.zeros((256, 896), _np.float32)
    for j in range(7):
        for c in range(64):
            sel[2 * c + j, j * 128 + c] = 1.0
    sel = jnp.asarray(sel, jnp.bfloat16)
    patches = _patchify(xpad, sel).reshape(N, 147, 4096)
    x = _stem(patches, params["conv1_w"], params["conv1_b"])

    nb_layer = {0: 2, 1: 4, 2: 8, 3: 16}
    for li, blocks in enumerate(params["layers"]):
        x = _res_layer(x, blocks, stride=1 if li == 0 else 2,
                       nb=nb_layer[li])

    feat = x.reshape(N, -1)
    y = _fc(feat, params["fc_w"], params["fc_b"])
    nrm = jnp.sqrt(jnp.sum(y * y, axis=-1, keepdims=True))
    return y / jnp.maximum(nrm, 1e-12) * 10.0


def kernel(conv1_w, conv1_b,
           L0b0_w1, L0b0_b1, L0b0_w2, L0b0_b2, L0b0_w3, L0b0_b3, L0b0_wd, L0b0_bd,
           L0b1_w1, L0b1_b1, L0b1_w2, L0b1_b2, L0b1_w3, L0b1_b3,
           L0b2_w1, L0b2_b1, L0b2_w2, L0b2_b2, L0b2_w3, L0b2_b3,
           L1b0_w1, L1b0_b1, L1b0_w2, L1b0_b2, L1b0_w3, L1b0_b3, L1b0_wd, L1b0_bd,
           L1b1_w1, L1b1_b1, L1b1_w2, L1b1_b2, L1b1_w3, L1b1_b3,
           L1b2_w1, L1b2_b1, L1b2_w2, L1b2_b2, L1b2_w3, L1b2_b3,
           L1b3_w1, L1b3_b1, L1b3_w2, L1b3_b2, L1b3_w3, L1b3_b3,
           L2b0_w1, L2b0_b1, L2b0_w2, L2b0_b2, L2b0_w3, L2b0_b3, L2b0_wd, L2b0_bd,
           L2b1_w1, L2b1_b1, L2b1_w2, L2b1_b2, L2b1_w3, L2b1_b3,
           L2b2_w1, L2b2_b1, L2b2_w2, L2b2_b2, L2b2_w3, L2b2_b3,
           L2b3_w1, L2b3_b1, L2b3_w2, L2b3_b2, L2b3_w3, L2b3_b3,
           L2b4_w1, L2b4_b1, L2b4_w2, L2b4_b2, L2b4_w3, L2b4_b3,
           L2b5_w1, L2b5_b1, L2b5_w2, L2b5_b2, L2b5_w3, L2b5_b3,
           L3b0_w1, L3b0_b1, L3b0_w2, L3b0_b2, L3b0_w3, L3b0_b3, L3b0_wd, L3b0_bd,
           L3b1_w1, L3b1_b1, L3b1_w2, L3b1_b2, L3b1_w3, L3b1_b3,
           L3b2_w1, L3b2_b1, L3b2_w2, L3b2_b2, L3b2_w3, L3b2_b3,
           fc_w, fc_b, x):
    params = {
        "conv1_w": conv1_w, "conv1_b": conv1_b,
        "fc_w": fc_w, "fc_b": fc_b,
        "layers": [
            [
                {"w1": L0b0_w1, "b1": L0b0_b1, "w2": L0b0_w2, "b2": L0b0_b2,
                 "w3": L0b0_w3, "b3": L0b0_b3, "wd": L0b0_wd, "bd": L0b0_bd},
                {"w1": L0b1_w1, "b1": L0b1_b1, "w2": L0b1_w2, "b2": L0b1_b2,
                 "w3": L0b1_w3, "b3": L0b1_b3},
                {"w1": L0b2_w1, "b1": L0b2_b1, "w2": L0b2_w2, "b2": L0b2_b2,
                 "w3": L0b2_w3, "b3": L0b2_b3},
            ],
            [
                {"w1": L1b0_w1, "b1": L1b0_b1, "w2": L1b0_w2, "b2": L1b0_b2,
                 "w3": L1b0_w3, "b3": L1b0_b3, "wd": L1b0_wd, "bd": L1b0_bd},
                {"w1": L1b1_w1, "b1": L1b1_b1, "w2": L1b1_w2, "b2": L1b1_b2,
                 "w3": L1b1_w3, "b3": L1b1_b3},
                {"w1": L1b2_w1, "b1": L1b2_b1, "w2": L1b2_w2, "b2": L1b2_b2,
                 "w3": L1b2_w3, "b3": L1b2_b3},
                {"w1": L1b3_w1, "b1": L1b3_b1, "w2": L1b3_w2, "b2": L1b3_b2,
                 "w3": L1b3_w3, "b3": L1b3_b3},
            ],
            [
                {"w1": L2b0_w1, "b1": L2b0_b1, "w2": L2b0_w2, "b2": L2b0_b2,
                 "w3": L2b0_w3, "b3": L2b0_b3, "wd": L2b0_wd, "bd": L2b0_bd},
                {"w1": L2b1_w1, "b1": L2b1_b1, "w2": L2b1_w2, "b2": L2b1_b2,
                 "w3": L2b1_w3, "b3": L2b1_b3},
                {"w1": L2b2_w1, "b1": L2b2_b1, "w2": L2b2_w2, "b2": L2b2_b2,
                 "w3": L2b2_w3, "b3": L2b2_b3},
                {"w1": L2b3_w1, "b1": L2b3_b1, "w2": L2b3_w2, "b2": L2b3_b2,
                 "w3": L2b3_w3, "b3": L2b3_b3},
                {"w1": L2b4_w1, "b1": L2b4_b1, "w2": L2b4_w2, "b2": L2b4_b2,
                 "w3": L2b4_w3, "b3": L2b4_b3},
                {"w1": L2b5_w1, "b1": L2b5_b1, "w2": L2b5_w2, "b2": L2b5_b2,
                 "w3": L2b5_w3, "b3": L2b5_b3},
            ],
            [
                {"w1": L3b0_w1, "b1": L3b0_b1, "w2": L3b0_w2, "b2": L3b0_b2,
                 "w3": L3b0_w3, "b3": L3b0_b3, "wd": L3b0_wd, "bd": L3b0_bd},
                {"w1": L3b1_w1, "b1": L3b1_b1, "w2": L3b1_w2, "b2": L3b1_b2,
                 "w3": L3b1_w3, "b3": L3b1_b3},
                {"w1": L3b2_w1, "b1": L3b2_b1, "w2": L3b2_w2, "b2": L3b2_b2,
                 "w3": L3b2_w3, "b3": L3b2_b3},
            ],
        ],
    }
    return _forward(params, x)
